# baseline (device time: 2010647 ns/iter reference)
import jax
import jax.numpy as jnp
from jax import lax
from jax.experimental import pallas as pl
from jax.experimental.pallas import tpu as pltpu

N_DEV = 32
N_DIRS = 1


def kernel(x, w_mat):
    m, k_loc = x.shape
    _, n_out = w_mat.shape
    chunk_m = m // N_DEV
    n_cols = n_out // N_DIRS
    total = 2 * N_DEV - 2

    def body(x_ref, w_ref, out_ref, *scratch):
        comms = scratch[0:N_DIRS]
        stages = scratch[N_DIRS:2 * N_DIRS]
        send_sems = scratch[2 * N_DIRS:3 * N_DIRS]
        recv_sems = scratch[3 * N_DIRS:4 * N_DIRS]
        copy_sems = scratch[4 * N_DIRS:5 * N_DIRS]
        credit_sems = scratch[5 * N_DIRS:6 * N_DIRS]
        exit_sem = scratch[6 * N_DIRS]

        my = lax.axis_index("i")
        right = (my + 1) % N_DEV
        left = (my - 1) % N_DEV

        barrier_sem = pltpu.get_barrier_semaphore()
        for nbr in (left, right):
            pl.semaphore_signal(
                barrier_sem, inc=1, device_id=(nbr,),
                device_id_type=pl.DeviceIdType.MESH,
            )
        pl.semaphore_wait(barrier_sem, 2)

        def dir_params(d):
            col0 = d * n_cols
            dst = right if d == 0 else left
            crd = left if d == 0 else right
            sgn = -1 if d == 0 else 1
            return col0, dst, crd, sgn

        def partial_for(c, col0):
            xc = x_ref[pl.ds(c * chunk_m, chunk_m), :]
            wc = w_ref[:, col0:col0 + n_cols]
            return jnp.dot(xc, wc, preferred_element_type=jnp.float32)

        for d in range(N_DIRS):
            col0, _, _, _ = dir_params(d)
            comms[d][0, :, :] = partial_for(my, col0).astype(jnp.bfloat16)

        for u in range(total):
            s_slot = u % 2
            r_slot = (u + 1) % 2
            rdmas = []
            for d in range(N_DIRS):
                col0, dst, crd, sgn = dir_params(d)
                if u >= 1:
                    pl.semaphore_wait(credit_sems[d].at[r_slot], 1)
                rdma = pltpu.make_async_remote_copy(
                    src_ref=comms[d].at[s_slot],
                    dst_ref=comms[d].at[r_slot],
                    send_sem=send_sems[d].at[s_slot],
                    recv_sem=recv_sems[d].at[r_slot],
                    device_id=(dst,),
                    device_id_type=pl.DeviceIdType.MESH,
                )
                rdma.start()
                rdmas.append(rdma)

            parts = []
            if u < N_DEV - 1:
                for d in range(N_DIRS):
                    col0, _, _, sgn = dir_params(d)
                    c_in = (my + sgn * (u + 1)) % N_DEV
                    parts.append(partial_for(c_in, col0))

            for d in range(N_DIRS):
                col0, dst, crd, sgn = dir_params(d)
                rdmas[d].wait_recv()
                if u < N_DEV - 1:
                    acc = comms[d][r_slot, :, :].astype(jnp.float32) + parts[d]
                    comms[d][r_slot, :, :] = acc.astype(jnp.bfloat16)
                    if u == N_DEV - 2:
                        c_own = (my - sgn) % N_DEV
                        stages[d][:, :] = acc
                        cp = pltpu.make_async_copy(
                            stages[d],
                            out_ref.at[pl.ds(c_own * chunk_m, chunk_m),
                                       pl.ds(col0, n_cols)],
                            copy_sems[d],
                        )
                        cp.start()
                        cp.wait()
                else:
                    g = u - (N_DEV - 1)
                    c_got = (my + sgn * g) % N_DEV
                    stages[d][:, :] = comms[d][r_slot, :, :].astype(jnp.float32)
                    cp = pltpu.make_async_copy(
                        stages[d],
                        out_ref.at[pl.ds(c_got * chunk_m, chunk_m),
                                   pl.ds(col0, n_cols)],
                        copy_sems[d],
                    )
                    cp.start()
                    cp.wait()

            for d in range(N_DIRS):
                col0, dst, crd, sgn = dir_params(d)
                rdmas[d].wait_send()
                if u <= total - 2:
                    pl.semaphore_signal(
                        credit_sems[d].at[s_slot], inc=1, device_id=(crd,),
                        device_id_type=pl.DeviceIdType.MESH,
                    )

        for nbr in (left, right):
            pl.semaphore_signal(
                exit_sem, inc=1, device_id=(nbr,),
                device_id_type=pl.DeviceIdType.MESH,
            )
        pl.semaphore_wait(exit_sem, 2)

    scratch = (
        [pltpu.VMEM((2, chunk_m, n_cols), jnp.bfloat16)] * N_DIRS
        + [pltpu.VMEM((chunk_m, n_cols), jnp.float32)] * N_DIRS
        + [pltpu.SemaphoreType.DMA((2,))] * N_DIRS
        + [pltpu.SemaphoreType.DMA((2,))] * N_DIRS
        + [pltpu.SemaphoreType.DMA] * N_DIRS
        + [pltpu.SemaphoreType.REGULAR((2,))] * N_DIRS
        + [pltpu.SemaphoreType.REGULAR]
    )

    return pl.pallas_call(
        body,
        out_shape=jax.ShapeDtypeStruct((m, n_out), jnp.float32),
        in_specs=[
            pl.BlockSpec(memory_space=pltpu.MemorySpace.VMEM),
            pl.BlockSpec(memory_space=pltpu.MemorySpace.VMEM),
        ],
        out_specs=pl.BlockSpec(memory_space=pltpu.MemorySpace.HBM),
        scratch_shapes=scratch,
        compiler_params=pltpu.CompilerParams(collective_id=0),
    )(x, w_mat)


# device time: 1691866 ns/iter; 1.1884x vs baseline; 1.1884x over previous
import jax
import jax.numpy as jnp
from jax import lax
from jax.experimental import pallas as pl
from jax.experimental.pallas import tpu as pltpu

N_DEV = 32
N_DIRS = 2


def kernel(x, w_mat):
    m, k_loc = x.shape
    _, n_out = w_mat.shape
    chunk_m = m // N_DEV
    n_cols = n_out // N_DIRS
    total = 2 * N_DEV - 2

    def body(x_ref, w_ref, out_ref, *scratch):
        comms = scratch[0:N_DIRS]
        stages = scratch[N_DIRS:2 * N_DIRS]
        send_sems = scratch[2 * N_DIRS:3 * N_DIRS]
        recv_sems = scratch[3 * N_DIRS:4 * N_DIRS]
        copy_sems = scratch[4 * N_DIRS:5 * N_DIRS]
        credit_sems = scratch[5 * N_DIRS:6 * N_DIRS]
        exit_sem = scratch[6 * N_DIRS]

        my = lax.axis_index("i")
        right = (my + 1) % N_DEV
        left = (my - 1) % N_DEV

        barrier_sem = pltpu.get_barrier_semaphore()
        for nbr in (left, right):
            pl.semaphore_signal(
                barrier_sem, inc=1, device_id=(nbr,),
                device_id_type=pl.DeviceIdType.MESH,
            )
        pl.semaphore_wait(barrier_sem, 2)

        def dir_params(d):
            col0 = d * n_cols
            dst = right if d == 0 else left
            crd = left if d == 0 else right
            sgn = -1 if d == 0 else 1
            return col0, dst, crd, sgn

        def partial_for(c, col0):
            xc = x_ref[pl.ds(c * chunk_m, chunk_m), :]
            wc = w_ref[:, col0:col0 + n_cols]
            return jnp.dot(xc, wc, preferred_element_type=jnp.float32)

        for d in range(N_DIRS):
            col0, _, _, _ = dir_params(d)
            comms[d][0, :, :] = partial_for(my, col0).astype(jnp.bfloat16)

        for u in range(total):
            s_slot = u % 2
            r_slot = (u + 1) % 2
            rdmas = []
            for d in range(N_DIRS):
                col0, dst, crd, sgn = dir_params(d)
                if u >= 1:
                    pl.semaphore_wait(credit_sems[d].at[r_slot], 1)
                rdma = pltpu.make_async_remote_copy(
                    src_ref=comms[d].at[s_slot],
                    dst_ref=comms[d].at[r_slot],
                    send_sem=send_sems[d].at[s_slot],
                    recv_sem=recv_sems[d].at[r_slot],
                    device_id=(dst,),
                    device_id_type=pl.DeviceIdType.MESH,
                )
                rdma.start()
                rdmas.append(rdma)

            parts = []
            if u < N_DEV - 1:
                for d in range(N_DIRS):
                    col0, _, _, sgn = dir_params(d)
                    c_in = (my + sgn * (u + 1)) % N_DEV
                    parts.append(partial_for(c_in, col0))

            for d in range(N_DIRS):
                col0, dst, crd, sgn = dir_params(d)
                rdmas[d].wait_recv()
                if u < N_DEV - 1:
                    acc = comms[d][r_slot, :, :].astype(jnp.float32) + parts[d]
                    comms[d][r_slot, :, :] = acc.astype(jnp.bfloat16)
                    if u == N_DEV - 2:
                        c_own = (my - sgn) % N_DEV
                        stages[d][:, :] = acc
                        cp = pltpu.make_async_copy(
                            stages[d],
                            out_ref.at[pl.ds(c_own * chunk_m, chunk_m),
                                       pl.ds(col0, n_cols)],
                            copy_sems[d],
                        )
                        cp.start()
                        cp.wait()
                else:
                    g = u - (N_DEV - 1)
                    c_got = (my + sgn * g) % N_DEV
                    stages[d][:, :] = comms[d][r_slot, :, :].astype(jnp.float32)
                    cp = pltpu.make_async_copy(
                        stages[d],
                        out_ref.at[pl.ds(c_got * chunk_m, chunk_m),
                                   pl.ds(col0, n_cols)],
                        copy_sems[d],
                    )
                    cp.start()
                    cp.wait()

            for d in range(N_DIRS):
                col0, dst, crd, sgn = dir_params(d)
                rdmas[d].wait_send()
                if u <= total - 2:
                    pl.semaphore_signal(
                        credit_sems[d].at[s_slot], inc=1, device_id=(crd,),
                        device_id_type=pl.DeviceIdType.MESH,
                    )

        for nbr in (left, right):
            pl.semaphore_signal(
                exit_sem, inc=1, device_id=(nbr,),
                device_id_type=pl.DeviceIdType.MESH,
            )
        pl.semaphore_wait(exit_sem, 2)

    scratch = (
        [pltpu.VMEM((2, chunk_m, n_cols), jnp.bfloat16)] * N_DIRS
        + [pltpu.VMEM((chunk_m, n_cols), jnp.float32)] * N_DIRS
        + [pltpu.SemaphoreType.DMA((2,))] * N_DIRS
        + [pltpu.SemaphoreType.DMA((2,))] * N_DIRS
        + [pltpu.SemaphoreType.DMA] * N_DIRS
        + [pltpu.SemaphoreType.REGULAR((2,))] * N_DIRS
        + [pltpu.SemaphoreType.REGULAR]
    )

    return pl.pallas_call(
        body,
        out_shape=jax.ShapeDtypeStruct((m, n_out), jnp.float32),
        in_specs=[
            pl.BlockSpec(memory_space=pltpu.MemorySpace.VMEM),
            pl.BlockSpec(memory_space=pltpu.MemorySpace.VMEM),
        ],
        out_specs=pl.BlockSpec(memory_space=pltpu.MemorySpace.HBM),
        scratch_shapes=scratch,
        compiler_params=pltpu.CompilerParams(collective_id=0),
    )(x, w_mat)


# device time: 1047828 ns/iter; 1.9189x vs baseline; 1.6146x over previous
import jax
import jax.numpy as jnp
from jax import lax
from jax.experimental import pallas as pl
from jax.experimental.pallas import tpu as pltpu

N_DEV = 32
N_DIRS = 2


def kernel(x, w_mat):
    m, k_loc = x.shape
    _, n_out = w_mat.shape
    chunk_m = m // N_DEV
    n_cols = n_out // N_DIRS
    total = 2 * N_DEV - 2

    def body(x_ref, w_ref, out_ref, *scratch):
        comms = scratch[0:N_DIRS]
        stages = scratch[N_DIRS:2 * N_DIRS]
        send_sems = scratch[2 * N_DIRS:3 * N_DIRS]
        recv_sems = scratch[3 * N_DIRS:4 * N_DIRS]
        copy_sems = scratch[4 * N_DIRS:5 * N_DIRS]
        credit_sems = scratch[5 * N_DIRS:6 * N_DIRS]
        exit_sem = scratch[6 * N_DIRS]

        def pos_of(dev):
            z = dev // 8
            o = dev % 8
            x = (o ^ (o >> 1)) & 1
            y = o // 2
            s = 4 * y + jnp.where(y % 2 == 0, z, 3 - z)
            return jnp.where(x == 0, s, 31 - s)

        def dev_of(p):
            p = p % N_DEV
            x = p // 16
            s = jnp.where(p < 16, p, 31 - p)
            y = s // 4
            zp = s % 4
            z = jnp.where(y % 2 == 0, zp, 3 - zp)
            t = jnp.where(y % 2 == 0, x, 1 - x)
            return 8 * z + 2 * y + t

        my = pos_of(lax.axis_index("i"))
        right = dev_of(my + 1)
        left = dev_of(my - 1)

        barrier_sem = pltpu.get_barrier_semaphore()
        for nbr in (left, right):
            pl.semaphore_signal(
                barrier_sem, inc=1, device_id=(nbr,),
                device_id_type=pl.DeviceIdType.MESH,
            )
        pl.semaphore_wait(barrier_sem, 2)

        def dir_params(d):
            col0 = d * n_cols
            dst = right if d == 0 else left
            crd = left if d == 0 else right
            sgn = -1 if d == 0 else 1
            return col0, dst, crd, sgn

        def partial_for(c, col0):
            xc = x_ref[pl.ds(c * chunk_m, chunk_m), :]
            wc = w_ref[:, col0:col0 + n_cols]
            return jnp.dot(xc, wc, preferred_element_type=jnp.float32)

        for d in range(N_DIRS):
            col0, _, _, _ = dir_params(d)
            comms[d][0, :, :] = partial_for(my, col0).astype(jnp.bfloat16)

        for u in range(total):
            s_slot = u % 2
            r_slot = (u + 1) % 2
            rdmas = []
            for d in range(N_DIRS):
                col0, dst, crd, sgn = dir_params(d)
                if u >= 1:
                    pl.semaphore_wait(credit_sems[d].at[r_slot], 1)
                rdma = pltpu.make_async_remote_copy(
                    src_ref=comms[d].at[s_slot],
                    dst_ref=comms[d].at[r_slot],
                    send_sem=send_sems[d].at[s_slot],
                    recv_sem=recv_sems[d].at[r_slot],
                    device_id=(dst,),
                    device_id_type=pl.DeviceIdType.MESH,
                )
                rdma.start()
                rdmas.append(rdma)

            parts = []
            if u < N_DEV - 1:
                for d in range(N_DIRS):
                    col0, _, _, sgn = dir_params(d)
                    c_in = (my + sgn * (u + 1)) % N_DEV
                    parts.append(partial_for(c_in, col0))

            for d in range(N_DIRS):
                col0, dst, crd, sgn = dir_params(d)
                rdmas[d].wait_recv()
                if u < N_DEV - 1:
                    acc = comms[d][r_slot, :, :].astype(jnp.float32) + parts[d]
                    comms[d][r_slot, :, :] = acc.astype(jnp.bfloat16)
                    if u == N_DEV - 2:
                        c_own = (my - sgn) % N_DEV
                        stages[d][:, :] = acc
                        cp = pltpu.make_async_copy(
                            stages[d],
                            out_ref.at[pl.ds(c_own * chunk_m, chunk_m),
                                       pl.ds(col0, n_cols)],
                            copy_sems[d],
                        )
                        cp.start()
                        cp.wait()
                else:
                    g = u - (N_DEV - 1)
                    c_got = (my + sgn * g) % N_DEV
                    stages[d][:, :] = comms[d][r_slot, :, :].astype(jnp.float32)
                    cp = pltpu.make_async_copy(
                        stages[d],
                        out_ref.at[pl.ds(c_got * chunk_m, chunk_m),
                                   pl.ds(col0, n_cols)],
                        copy_sems[d],
                    )
                    cp.start()
                    cp.wait()

            for d in range(N_DIRS):
                col0, dst, crd, sgn = dir_params(d)
                rdmas[d].wait_send()
                if u <= total - 2:
                    pl.semaphore_signal(
                        credit_sems[d].at[s_slot], inc=1, device_id=(crd,),
                        device_id_type=pl.DeviceIdType.MESH,
                    )

        for nbr in (left, right):
            pl.semaphore_signal(
                exit_sem, inc=1, device_id=(nbr,),
                device_id_type=pl.DeviceIdType.MESH,
            )
        pl.semaphore_wait(exit_sem, 2)

    scratch = (
        [pltpu.VMEM((2, chunk_m, n_cols), jnp.bfloat16)] * N_DIRS
        + [pltpu.VMEM((chunk_m, n_cols), jnp.float32)] * N_DIRS
        + [pltpu.SemaphoreType.DMA((2,))] * N_DIRS
        + [pltpu.SemaphoreType.DMA((2,))] * N_DIRS
        + [pltpu.SemaphoreType.DMA] * N_DIRS
        + [pltpu.SemaphoreType.REGULAR((2,))] * N_DIRS
        + [pltpu.SemaphoreType.REGULAR]
    )

    return pl.pallas_call(
        body,
        out_shape=jax.ShapeDtypeStruct((m, n_out), jnp.float32),
        in_specs=[
            pl.BlockSpec(memory_space=pltpu.MemorySpace.VMEM),
            pl.BlockSpec(memory_space=pltpu.MemorySpace.VMEM),
        ],
        out_specs=pl.BlockSpec(memory_space=pltpu.MemorySpace.HBM),
        scratch_shapes=scratch,
        compiler_params=pltpu.CompilerParams(collective_id=0),
    )(x, w_mat)
